# Initial kernel scaffold; baseline (speedup 1.0000x reference)
import jax
import jax.numpy as jnp
from jax.experimental import pallas as pl


def kernel(v_coords, v_weights, simp1_verts, simp1_weights, simp2_verts, simp2_weights, dirs):
    raise NotImplementedError("write your pallas kernel here")


# speedup vs baseline: 27.1155x; 27.1155x over previous
import functools

import jax
import jax.numpy as jnp
from jax import lax
from jax.experimental import pallas as pl
from jax.experimental.pallas import tpu as pltpu
from jax.experimental.pallas import tpu_sc as plsc

NUM_H = 256
DPAD = 32
NW = 32
BLK = 128
HIST = DPAD * NUM_H


def _ceil_to(x: int, m: int) -> int:
    return (x + m - 1) // m * m


def _maxnorm2_body(c_ref, o_ref):
    i = pl.program_id(0)
    x = c_ref[...]
    m = jnp.max(jnp.sum(x * x, axis=1))

    @pl.when(i == 0)
    def _():
        o_ref[0, 0] = m

    @pl.when(i > 0)
    def _():
        o_ref[0, 0] = jnp.maximum(o_ref[0, 0], m)


def _bins_body(mh2_ref, dT_ref, c_ref, o_ref):
    eps = 1e-12
    mh = jnp.maximum(jnp.sqrt(mh2_ref[0, 0]), eps)
    dT = dT_ref[...]
    n = jnp.sqrt(jnp.sum(dT * dT, axis=0, keepdims=True))
    dn = dT / jnp.maximum(n, eps)
    c = c_ref[...]
    h = (c[:, 0:1] * dn[0:1, :] + c[:, 1:2] * dn[1:2, :]
         + c[:, 2:3] * dn[2:3, :])
    u = (NUM_H - 1) * (mh + h) / (2.0 * mh)
    o_ref[...] = jnp.clip(jnp.ceil(u), 0, NUM_H - 1).astype(jnp.int32)


def _sc_wect(table, ea, eb, ew, ta, tb, tc, tw, vw,
             eb_blocks, tb_blocks, vb_blocks):
    mesh = plsc.VectorSubcoreMesh(core_axis_name="c", subcore_axis_name="s")

    @functools.partial(
        pl.kernel, mesh=mesh,
        out_type=jax.ShapeDtypeStruct((NW, HIST), jnp.float32),
        compiler_params=pltpu.CompilerParams(needs_layout_passes=False,
                                             use_tc_tiling_on_sc=False),
        scratch_types=[
            pltpu.VMEM((BLK,), jnp.int32),
            pltpu.VMEM((BLK,), jnp.int32),
            pltpu.VMEM((BLK,), jnp.int32),
            pltpu.VMEM((BLK, DPAD), jnp.int32),
            pltpu.VMEM((BLK, DPAD), jnp.int32),
            pltpu.VMEM((BLK, DPAD), jnp.int32),
            pltpu.VMEM((BLK,), jnp.float32),
            pltpu.VMEM((HIST,), jnp.float32),
            pltpu.SemaphoreType.DMA,
        ],
    )
    def k(table_h, ea_h, eb_h, ta_h, tb_h, tc_h, ew_h, tw_h, vw_h, out_h,
          ia, ib, ic, ra, rb, rc, wb, hist, sem):
        wid = lax.axis_index("s") * 2 + lax.axis_index("c")

        lanes = lax.iota(jnp.int32, 16)
        offs = [lanes * NUM_H, lanes * NUM_H + 16 * NUM_H]

        def zero_body(i, _):
            hist[pl.ds(i * 16, 16)] = jnp.zeros((16,), jnp.float32)
            return 0

        lax.fori_loop(0, HIST // 16, zero_body, 0)

        def inner(row_refs, sign):
            def body(g, _):
                w16 = wb[pl.ds(g * 16, 16)]
                if sign < 0:
                    w16 = -w16
                for j in range(16):
                    wv = jnp.full((16,), w16[j], dtype=jnp.float32)
                    s = g * 16 + j
                    for half in range(2):
                        sl = pl.ds(half * 16, 16)
                        m = row_refs[0][s, sl]
                        for r in row_refs[1:]:
                            m = jnp.maximum(m, r[s, sl])
                        plsc.addupdate_scatter(hist, [m + offs[half]], wv)
                return 0

            lax.fori_loop(0, BLK // 16, body, 0)

        def simplex_phase(nblk, id_hs, id_vs, row_vs, w_h, sign):
            per_w = nblk * BLK

            def blk(i, _):
                base = wid * per_w + i * BLK
                cps = [pltpu.async_copy(h.at[pl.ds(base, BLK)], v, sem)
                       for h, v in zip(id_hs, id_vs)]
                cps.append(pltpu.async_copy(w_h.at[pl.ds(base, BLK)], wb, sem))
                for c in cps:
                    c.wait()
                gps = [pltpu.async_copy(table_h.at[iv], rv, sem)
                       for iv, rv in zip(id_vs, row_vs)]
                for g in gps:
                    g.wait()
                inner(row_vs, sign)
                return 0

            lax.fori_loop(0, nblk, blk, 0)

        def vblk(i, _):
            base = wid * (vb_blocks * BLK) + i * BLK
            c0 = pltpu.async_copy(table_h.at[pl.ds(base, BLK)], ra, sem)
            c1 = pltpu.async_copy(vw_h.at[pl.ds(base, BLK)], wb, sem)
            c0.wait()
            c1.wait()
            inner([ra], 1)
            return 0

        lax.fori_loop(0, vb_blocks, vblk, 0)
        simplex_phase(eb_blocks, [ea_h, eb_h], [ia, ib], [ra, rb], ew_h, -1)
        simplex_phase(tb_blocks, [ta_h, tb_h, tc_h], [ia, ib, ic],
                      [ra, rb, rc], tw_h, 1)

        pltpu.sync_copy(hist, out_h.at[wid])

    return k(table, ea, eb, ta, tb, tc, ew, tw, vw)


def _finish_body(p_ref, o_ref):
    s = jnp.sum(p_ref[...], axis=0)
    r = lax.broadcasted_iota(jnp.int32, (NUM_H, NUM_H), 0)
    c = lax.broadcasted_iota(jnp.int32, (NUM_H, NUM_H), 1)
    tri = (r <= c).astype(jnp.float32)
    cum = lax.dot_general(s, tri, (((1,), (0,)), ((), ())),
                          precision=lax.Precision.HIGHEST,
                          preferred_element_type=jnp.float32)
    o_ref[...] = cum[:26, :]


def kernel(v_coords, v_weights, simp1_verts, simp1_weights, simp2_verts,
           simp2_weights, dirs):
    nv = v_coords.shape[0]
    ne = simp1_verts.shape[0]
    nt = simp2_verts.shape[0]

    vb_blocks = _ceil_to(nv, NW * BLK) // (NW * BLK)
    eb_blocks = _ceil_to(ne, NW * BLK) // (NW * BLK)
    tb_blocks = _ceil_to(nt, NW * BLK) // (NW * BLK)
    vp = vb_blocks * NW * BLK
    ep = eb_blocks * NW * BLK
    tp = tb_blocks * NW * BLK

    coords = jnp.pad(v_coords, ((0, vp - nv), (0, 0)))
    vw = jnp.pad(v_weights, (0, vp - nv))
    ea = jnp.pad(simp1_verts[:, 0], (0, ep - ne))
    eb = jnp.pad(simp1_verts[:, 1], (0, ep - ne))
    ew = jnp.pad(simp1_weights, (0, ep - ne))
    ta = jnp.pad(simp2_verts[:, 0], (0, tp - nt))
    tb = jnp.pad(simp2_verts[:, 1], (0, tp - nt))
    tc = jnp.pad(simp2_verts[:, 2], (0, tp - nt))
    tw = jnp.pad(simp2_weights, (0, tp - nt))
    dirsT = jnp.pad(dirs.T, ((0, 0), (0, DPAD - dirs.shape[0])))

    ab = 2048
    mh2 = pl.pallas_call(
        _maxnorm2_body,
        grid=(vp // ab,),
        in_specs=[pl.BlockSpec((ab, 3), lambda i: (i, 0))],
        out_specs=pl.BlockSpec(memory_space=pltpu.SMEM),
        out_shape=jax.ShapeDtypeStruct((1, 1), jnp.float32),
    )(coords)

    bb = 1024
    table = pl.pallas_call(
        _bins_body,
        grid=(vp // bb,),
        in_specs=[
            pl.BlockSpec(memory_space=pltpu.SMEM),
            pl.BlockSpec((3, DPAD), lambda i: (0, 0)),
            pl.BlockSpec((bb, 3), lambda i: (i, 0)),
        ],
        out_specs=pl.BlockSpec((bb, DPAD), lambda i: (i, 0)),
        out_shape=jax.ShapeDtypeStruct((vp, DPAD), jnp.int32),
    )(mh2, dirsT, coords)

    partials = _sc_wect(table, ea, eb, ew, ta, tb, tc, tw, vw,
                        eb_blocks, tb_blocks, vb_blocks)

    out = pl.pallas_call(
        _finish_body,
        in_specs=[pl.BlockSpec((NW, DPAD, NUM_H), lambda: (0, 0, 0))],
        out_specs=pl.BlockSpec((26, NUM_H), lambda: (0, 0)),
        out_shape=jax.ShapeDtypeStruct((26, NUM_H), jnp.float32),
    )(partials.reshape(NW, DPAD, NUM_H))
    return out


# speedup vs baseline: 34.9160x; 1.2877x over previous
import functools

import jax
import jax.numpy as jnp
from jax import lax
from jax.experimental import pallas as pl
from jax.experimental.pallas import tpu as pltpu
from jax.experimental.pallas import tpu_sc as plsc

NUM_H = 256
DPAD = 32
NW = 32
BLK = 128
HIST = DPAD * NUM_H


def _ceil_to(x: int, m: int) -> int:
    return (x + m - 1) // m * m


def _maxnorm2_body(c_ref, o_ref):
    i = pl.program_id(0)
    x = c_ref[...]
    m = jnp.max(jnp.sum(x * x, axis=1))

    @pl.when(i == 0)
    def _():
        o_ref[0, 0] = m

    @pl.when(i > 0)
    def _():
        o_ref[0, 0] = jnp.maximum(o_ref[0, 0], m)


def _bins_body(mh2_ref, dT_ref, c_ref, o_ref):
    eps = 1e-12
    mh = jnp.maximum(jnp.sqrt(mh2_ref[0, 0]), eps)
    dT = dT_ref[...]
    n = jnp.sqrt(jnp.sum(dT * dT, axis=0, keepdims=True))
    dn = dT / jnp.maximum(n, eps)
    c = c_ref[...]
    h = (c[:, 0:1] * dn[0:1, :] + c[:, 1:2] * dn[1:2, :]
         + c[:, 2:3] * dn[2:3, :])
    u = (NUM_H - 1) * (mh + h) / (2.0 * mh)
    o_ref[...] = jnp.clip(jnp.ceil(u), 0, NUM_H - 1).astype(jnp.int32)


def _sc_wect(table, ea, eb, ew, ta, tb, tc, tw, vw,
             eb_blocks, tb_blocks, vb_blocks):
    mesh = plsc.VectorSubcoreMesh(core_axis_name="c", subcore_axis_name="s")

    @functools.partial(
        pl.kernel, mesh=mesh,
        out_type=jax.ShapeDtypeStruct((NW, HIST), jnp.float32),
        compiler_params=pltpu.CompilerParams(needs_layout_passes=False,
                                             use_tc_tiling_on_sc=False),
        scratch_types=[
            pltpu.VMEM((BLK,), jnp.int32),
            pltpu.VMEM((BLK,), jnp.int32),
            pltpu.VMEM((BLK,), jnp.int32),
            pltpu.VMEM((BLK, DPAD), jnp.int32),
            pltpu.VMEM((BLK, DPAD), jnp.int32),
            pltpu.VMEM((BLK, DPAD), jnp.int32),
            pltpu.VMEM((BLK,), jnp.float32),
            pltpu.VMEM((HIST,), jnp.float32),
            pltpu.SemaphoreType.DMA,
        ],
    )
    def k(table_h, ea_h, eb_h, ta_h, tb_h, tc_h, ew_h, tw_h, vw_h, out_h,
          ia, ib, ic, ra, rb, rc, wb, hist, sem):
        wid = lax.axis_index("s") * 2 + lax.axis_index("c")

        lanes = lax.iota(jnp.int32, 16)
        offs = [lanes * NUM_H, lanes * NUM_H + 16 * NUM_H]

        @plsc.parallel_loop(0, HIST // 16)
        def _(i):
            hist[pl.ds(i * 16, 16)] = jnp.zeros((16,), jnp.float32)

        def inner(row_refs, sign):
            @plsc.parallel_loop(0, BLK // 16, unroll=2)
            def _(g):
                w16 = wb[pl.ds(g * 16, 16)]
                if sign < 0:
                    w16 = -w16
                for j in range(16):
                    wv = jnp.full((16,), w16[j], dtype=jnp.float32)
                    s = g * 16 + j
                    for half in range(2):
                        sl = pl.ds(half * 16, 16)
                        m = row_refs[0][s, sl]
                        for r in row_refs[1:]:
                            m = jnp.maximum(m, r[s, sl])
                        plsc.addupdate_scatter(hist, [m + offs[half]], wv)

        def simplex_phase(nblk, id_hs, id_vs, row_vs, w_h, sign):
            per_w = nblk * BLK

            def blk(i, _):
                base = wid * per_w + i * BLK
                cps = [pltpu.async_copy(h.at[pl.ds(base, BLK)], v, sem)
                       for h, v in zip(id_hs, id_vs)]
                cps.append(pltpu.async_copy(w_h.at[pl.ds(base, BLK)], wb, sem))
                for c in cps:
                    c.wait()
                gps = [pltpu.async_copy(table_h.at[iv], rv, sem)
                       for iv, rv in zip(id_vs, row_vs)]
                for g in gps:
                    g.wait()
                inner(row_vs, sign)
                return 0

            lax.fori_loop(0, nblk, blk, 0)

        def vblk(i, _):
            base = wid * (vb_blocks * BLK) + i * BLK
            c0 = pltpu.async_copy(table_h.at[pl.ds(base, BLK)], ra, sem)
            c1 = pltpu.async_copy(vw_h.at[pl.ds(base, BLK)], wb, sem)
            c0.wait()
            c1.wait()
            inner([ra], 1)
            return 0

        lax.fori_loop(0, vb_blocks, vblk, 0)
        simplex_phase(eb_blocks, [ea_h, eb_h], [ia, ib], [ra, rb], ew_h, -1)
        simplex_phase(tb_blocks, [ta_h, tb_h, tc_h], [ia, ib, ic],
                      [ra, rb, rc], tw_h, 1)

        pltpu.sync_copy(hist, out_h.at[wid])

    return k(table, ea, eb, ta, tb, tc, ew, tw, vw)


def _finish_body(p_ref, o_ref):
    s = jnp.sum(p_ref[...], axis=0)
    r = lax.broadcasted_iota(jnp.int32, (NUM_H, NUM_H), 0)
    c = lax.broadcasted_iota(jnp.int32, (NUM_H, NUM_H), 1)
    tri = (r <= c).astype(jnp.float32)
    cum = lax.dot_general(s, tri, (((1,), (0,)), ((), ())),
                          precision=lax.Precision.HIGHEST,
                          preferred_element_type=jnp.float32)
    o_ref[...] = cum[:26, :]


def kernel(v_coords, v_weights, simp1_verts, simp1_weights, simp2_verts,
           simp2_weights, dirs):
    nv = v_coords.shape[0]
    ne = simp1_verts.shape[0]
    nt = simp2_verts.shape[0]

    vb_blocks = _ceil_to(nv, NW * BLK) // (NW * BLK)
    eb_blocks = _ceil_to(ne, NW * BLK) // (NW * BLK)
    tb_blocks = _ceil_to(nt, NW * BLK) // (NW * BLK)
    vp = vb_blocks * NW * BLK
    ep = eb_blocks * NW * BLK
    tp = tb_blocks * NW * BLK

    coords = jnp.pad(v_coords, ((0, vp - nv), (0, 0)))
    vw = jnp.pad(v_weights, (0, vp - nv))
    ea = jnp.pad(simp1_verts[:, 0], (0, ep - ne))
    eb = jnp.pad(simp1_verts[:, 1], (0, ep - ne))
    ew = jnp.pad(simp1_weights, (0, ep - ne))
    ta = jnp.pad(simp2_verts[:, 0], (0, tp - nt))
    tb = jnp.pad(simp2_verts[:, 1], (0, tp - nt))
    tc = jnp.pad(simp2_verts[:, 2], (0, tp - nt))
    tw = jnp.pad(simp2_weights, (0, tp - nt))
    dirsT = jnp.pad(dirs.T, ((0, 0), (0, DPAD - dirs.shape[0])))

    ab = 2048
    mh2 = pl.pallas_call(
        _maxnorm2_body,
        grid=(vp // ab,),
        in_specs=[pl.BlockSpec((ab, 3), lambda i: (i, 0))],
        out_specs=pl.BlockSpec(memory_space=pltpu.SMEM),
        out_shape=jax.ShapeDtypeStruct((1, 1), jnp.float32),
    )(coords)

    bb = 1024
    table = pl.pallas_call(
        _bins_body,
        grid=(vp // bb,),
        in_specs=[
            pl.BlockSpec(memory_space=pltpu.SMEM),
            pl.BlockSpec((3, DPAD), lambda i: (0, 0)),
            pl.BlockSpec((bb, 3), lambda i: (i, 0)),
        ],
        out_specs=pl.BlockSpec((bb, DPAD), lambda i: (i, 0)),
        out_shape=jax.ShapeDtypeStruct((vp, DPAD), jnp.int32),
    )(mh2, dirsT, coords)

    partials = _sc_wect(table, ea, eb, ew, ta, tb, tc, tw, vw,
                        eb_blocks, tb_blocks, vb_blocks)

    out = pl.pallas_call(
        _finish_body,
        in_specs=[pl.BlockSpec((NW, DPAD, NUM_H), lambda: (0, 0, 0))],
        out_specs=pl.BlockSpec((26, NUM_H), lambda: (0, 0)),
        out_shape=jax.ShapeDtypeStruct((26, NUM_H), jnp.float32),
    )(partials.reshape(NW, DPAD, NUM_H))
    return out
